# baseline (speedup 1.0000x reference)
import functools

import jax
import jax.numpy as jnp
from jax import lax
from jax.experimental import pallas as pl
from jax.experimental.pallas import tpu as pltpu
from jax.experimental.pallas import tpu_sc as plsc

HIDDEN = 1024
PATCH = 4
BS = 4
SEQ = 2048

NC = 2
NS = 16
NW = NC * NS

M_TOTAL = BS * SEQ
SLICES = 2
M_SLICE = M_TOTAL // SLICES
N_ROWS_S = PATCH * M_SLICE
ROWS_PER_W = N_ROWS_S // NW
CHUNK = 16
NBUF = 4
NCHUNKS = ROWS_PER_W // CHUNK
NQ = NCHUNKS // NBUF

MB = 1024
N_OUT = HIDDEN


def _gather_rows(idx, embs):
    mesh = plsc.VectorSubcoreMesh(core_axis_name="c", subcore_axis_name="s")

    @functools.partial(
        pl.kernel,
        out_type=jax.ShapeDtypeStruct((N_ROWS_S, HIDDEN), jnp.float32),
        mesh=mesh,
        cost_estimate=pl.CostEstimate(
            flops=0, transcendentals=0,
            bytes_accessed=2 * N_ROWS_S * HIDDEN * 4,
        ),
        scratch_types=[
            pltpu.VMEM((NCHUNKS, CHUNK), jnp.int32),
            [pltpu.VMEM((CHUNK, HIDDEN), jnp.float32) for _ in range(NBUF)],
            [pltpu.SemaphoreType.DMA for _ in range(NBUF)],
            [pltpu.SemaphoreType.DMA for _ in range(NBUF)],
        ],
    )
    def k(idx_hbm, table_hbm, out_hbm, idx_v, bufs, sgs, sws):
        wid = lax.axis_index("s") * NC + lax.axis_index("c")
        base = wid * ROWS_PER_W
        pltpu.sync_copy(idx_hbm.at[wid], idx_v)

        def gather(c, b):
            return pltpu.make_async_copy(table_hbm.at[idx_v.at[c]], bufs[b], sgs[b])

        def write(c, b):
            return pltpu.make_async_copy(
                bufs[b], out_hbm.at[pl.ds(base + c * CHUNK, CHUNK)], sws[b])

        for b in range(NBUF):
            gather(b, b).start()

        def body(i, carry):
            c0 = i * NBUF
            for b in range(NBUF):
                gather(c0 + b, b).wait()
                write(c0 + b, b).start()
            for b in range(NBUF):
                write(c0 + b, b).wait()
                gather(c0 + NBUF + b, b).start()
            return carry

        lax.fori_loop(0, NQ - 1, body, 0)

        cl = (NQ - 1) * NBUF
        for b in range(NBUF):
            gather(cl + b, b).wait()
            write(cl + b, b).start()
        for b in range(NBUF):
            write(cl + b, b).wait()

    return k(idx, embs)


def _matmul_kernel(prev_ref, hs_ref, w_ref, e0_ref, m_ref, o_ref):
    del prev_ref
    acc = lax.dot_general(
        hs_ref[0].astype(jnp.bfloat16), w_ref[0],
        (((1,), (0,)), ((), ())), preferred_element_type=jnp.float32,
    )
    for p in range(1, PATCH):
        acc += lax.dot_general(
            hs_ref[p].astype(jnp.bfloat16), w_ref[p],
            (((1,), (0,)), ((), ())), preferred_element_type=jnp.float32,
        )
    wsum = w_ref[0] + w_ref[1] + w_ref[2] + w_ref[3]
    v0 = lax.dot_general(
        e0_ref[...].astype(jnp.bfloat16), wsum,
        (((1,), (0,)), ((), ())), preferred_element_type=jnp.float32,
    )
    o_ref[...] = jnp.where(m_ref[...] != 0, v0, acc)


def _project_into(out_prev, hs_p, w_r, e0, mask_flat, s):
    grid = M_SLICE // MB
    off = s * grid
    return pl.pallas_call(
        _matmul_kernel,
        grid=(grid,),
        in_specs=[
            pl.BlockSpec(memory_space=pl.ANY),
            pl.BlockSpec((PATCH, MB, HIDDEN), lambda i: (0, i, 0)),
            pl.BlockSpec((PATCH, HIDDEN, N_OUT), lambda i: (0, 0, 0)),
            pl.BlockSpec((1, HIDDEN), lambda i: (0, 0)),
            pl.BlockSpec((MB, 1), lambda i, o=off: (o + i, 0)),
        ],
        out_specs=pl.BlockSpec((MB, N_OUT), lambda i, o=off: (o + i, 0)),
        out_shape=jax.ShapeDtypeStruct((M_TOTAL, N_OUT), jnp.float32),
        input_output_aliases={0: 0},
        cost_estimate=pl.CostEstimate(
            flops=2 * M_SLICE * PATCH * HIDDEN * N_OUT,
            transcendentals=0,
            bytes_accessed=PATCH * M_SLICE * HIDDEN * 4 + M_SLICE * N_OUT * 4,
        ),
    )(out_prev, hs_p, w_r, e0, mask_flat)


def kernel(input_ids, mask, embs, W_proj):
    idx_t = jnp.transpose(input_ids.astype(jnp.int32) + 1, (2, 0, 1))
    idx_t = idx_t.reshape(PATCH, M_TOTAL)
    w_r = jnp.transpose(
        W_proj.reshape(N_OUT, PATCH, HIDDEN), (1, 2, 0)
    ).astype(jnp.bfloat16)
    e0 = embs[0:1]
    mask_flat = mask.reshape(M_TOTAL, 1).astype(jnp.int32)

    hs_slices = []
    for s in range(SLICES):
        idx_s = idx_t[:, s * M_SLICE:(s + 1) * M_SLICE]
        idx_s = idx_s.reshape(NW, NCHUNKS, CHUNK)
        hs = _gather_rows(idx_s, embs)
        hs_slices.append(hs.reshape(PATCH, M_SLICE, HIDDEN))

    out = jnp.zeros((M_TOTAL, N_OUT), jnp.float32)
    for s in range(SLICES):
        out = _project_into(out, hs_slices[s], w_r, e0, mask_flat, s)
    return out.reshape(BS, SEQ, N_OUT)


# speedup vs baseline: 1.1184x; 1.1184x over previous
import functools

import jax
import jax.numpy as jnp
from jax import lax
from jax.experimental import pallas as pl
from jax.experimental.pallas import tpu as pltpu
from jax.experimental.pallas import tpu_sc as plsc

HIDDEN = 1024
PATCH = 4
BS = 4
SEQ = 2048

NC = 2
NS = 16
NW = NC * NS

M_TOTAL = BS * SEQ
N_ROWS = PATCH * M_TOTAL
ROWS_PER_W = N_ROWS // NW
CHUNK = 16
NCHUNKS = ROWS_PER_W // CHUNK
P = 2
NP = NCHUNKS // P

MB = 1024
N_OUT = HIDDEN


def _gather_rows(idx, embs):
    mesh = plsc.VectorSubcoreMesh(core_axis_name="c", subcore_axis_name="s")

    @functools.partial(
        pl.kernel,
        out_type=jax.ShapeDtypeStruct((N_ROWS, HIDDEN), jnp.float32),
        mesh=mesh,
        scratch_types=[
            pltpu.VMEM((NCHUNKS, CHUNK), jnp.int32),
            [pltpu.VMEM((CHUNK, HIDDEN), jnp.float32) for _ in range(2 * P)],
            [pltpu.SemaphoreType.DMA for _ in range(2 * P)],
            [pltpu.SemaphoreType.DMA for _ in range(2 * P)],
        ],
    )
    def k(idx_hbm, table_hbm, out_hbm, idx_v, bufs, sgs, sws):
        wid = lax.axis_index("s") * NC + lax.axis_index("c")
        base = wid * ROWS_PER_W
        pltpu.sync_copy(idx_hbm.at[wid], idx_v)

        def gather(c, b):
            return pltpu.make_async_copy(table_hbm.at[idx_v.at[c]], bufs[b], sgs[b])

        def write(c, b):
            return pltpu.make_async_copy(
                bufs[b], out_hbm.at[pl.ds(base + c * CHUNK, CHUNK)], sws[b])

        def phase_wait_g_start_w(j, grp):
            for b in range(P):
                gather(j * P + b, grp * P + b).wait()
                write(j * P + b, grp * P + b).start()

        def phase_wait_w_start_g(jw, jg, grp):
            for b in range(P):
                write(jw * P + b, grp * P + b).wait()
                gather(jg * P + b, grp * P + b).start()

        for b in range(P):
            gather(b, b).start()
        phase_wait_g_start_w(0, 0)
        for b in range(P):
            gather(P + b, P + b).start()
        phase_wait_g_start_w(1, 1)
        phase_wait_w_start_g(0, 2, 0)

        def body(i, carry):
            j = 2 * i
            phase_wait_g_start_w(j, 0)
            phase_wait_w_start_g(j - 1, j + 1, 1)
            phase_wait_g_start_w(j + 1, 1)
            phase_wait_w_start_g(j, j + 2, 0)
            return carry

        lax.fori_loop(1, NP // 2 - 1, body, 0)

        jl = NP - 2
        phase_wait_g_start_w(jl, 0)
        phase_wait_w_start_g(jl - 1, jl + 1, 1)
        phase_wait_g_start_w(jl + 1, 1)
        for b in range(P):
            write(jl * P + b, b).wait()
            write((jl + 1) * P + b, P + b).wait()

    return k(idx, embs)


def _matmul_kernel(hs_ref, w_ref, e0_ref, m_ref, o_ref, v0_ref):
    @pl.when(pl.program_id(0) == 0)
    def _():
        wsum = w_ref[0] + w_ref[1] + w_ref[2] + w_ref[3]
        v0_ref[...] = lax.dot_general(
            e0_ref[...].astype(jnp.bfloat16), wsum,
            (((1,), (0,)), ((), ())), preferred_element_type=jnp.float32,
        )

    acc = lax.dot_general(
        hs_ref[0].astype(jnp.bfloat16), w_ref[0],
        (((1,), (0,)), ((), ())), preferred_element_type=jnp.float32,
    )
    for p in range(1, PATCH):
        acc += lax.dot_general(
            hs_ref[p].astype(jnp.bfloat16), w_ref[p],
            (((1,), (0,)), ((), ())), preferred_element_type=jnp.float32,
        )
    o_ref[...] = jnp.where(m_ref[...] != 0, v0_ref[...], acc)


def _project(hs_p, w_r, e0, mask2):
    m = hs_p.shape[1]
    return pl.pallas_call(
        _matmul_kernel,
        grid=(m // MB,),
        in_specs=[
            pl.BlockSpec((PATCH, MB, HIDDEN), lambda i: (0, i, 0)),
            pl.BlockSpec((PATCH, HIDDEN, N_OUT), lambda i: (0, 0, 0)),
            pl.BlockSpec((1, HIDDEN), lambda i: (0, 0)),
            pl.BlockSpec((MB, 1), lambda i: (i, 0)),
        ],
        out_specs=pl.BlockSpec((MB, N_OUT), lambda i: (i, 0)),
        out_shape=jax.ShapeDtypeStruct((m, N_OUT), jnp.float32),
        scratch_shapes=[pltpu.VMEM((1, N_OUT), jnp.float32)],
    )(hs_p, w_r, e0, mask2)


def kernel(input_ids, mask, embs, W_proj):
    idx_t = jnp.transpose(input_ids.astype(jnp.int32) + 1, (2, 0, 1))
    idx = idx_t.reshape(NW, NCHUNKS, CHUNK)
    hs = _gather_rows(idx, embs)
    hs_p = hs.reshape(PATCH, M_TOTAL, HIDDEN)
    w_r = jnp.transpose(
        W_proj.reshape(N_OUT, PATCH, HIDDEN), (1, 2, 0)
    ).astype(jnp.bfloat16)
    mask2 = mask.reshape(M_TOTAL, 1).astype(jnp.int32)
    out = _project(hs_p, w_r, embs[0:1], mask2)
    return out.reshape(BS, SEQ, N_OUT)
